# baseline (speedup 1.0000x reference)
import functools

import jax
import jax.numpy as jnp
from jax import lax
from jax.experimental import pallas as pl
from jax.experimental.pallas import tpu as pltpu
from jax.experimental.pallas import tpu_sc as plsc

NC = 2
NS = 16
L = 16
NW = NC * NS

B = 16384
D = 32
GC = 64
BPW = B // NW
NSTEP = BPW // GC

_mesh = plsc.VectorSubcoreMesh(core_axis_name="c", subcore_axis_name="s")


@functools.partial(
    pl.kernel,
    mesh=_mesh,
    compiler_params=pltpu.CompilerParams(
        needs_layout_passes=False, use_tc_tiling_on_sc=True),
    out_type=jax.ShapeDtypeStruct((B,), jnp.float32),
    scratch_types=[
        pltpu.VMEM((BPW,), jnp.int32),
        pltpu.VMEM((BPW,), jnp.int32),
        pltpu.VMEM((2, GC, 1, D), jnp.float32),
        pltpu.VMEM((2, GC, 1, D), jnp.float32),
        pltpu.VMEM((2 * GC, D), jnp.float32),
        pltpu.VMEM((BPW,), jnp.float32),
        pltpu.SemaphoreType.DMA,
        pltpu.SemaphoreType.DMA,
    ],
)
def _sc_embed_dot(xt_hbm, tab3_hbm, out_hbm,
                  i0_v, i1_v, rows0_v, rows1_v, drain_v, out_v, sem_a, sem_b):
    wid = lax.axis_index("s") * NC + lax.axis_index("c")
    base = pl.multiple_of(wid * BPW, 128)
    tab_hbm = tab3_hbm.at[0]

    pltpu.sync_copy(xt_hbm.at[0, pl.ds(base, BPW)], i0_v)
    pltpu.sync_copy(xt_hbm.at[1, pl.ds(base, BPW)], i1_v)

    for g in range(BPW // L):
        a = i0_v[pl.ds(g * L, L)]
        b = i1_v[pl.ds(g * L, L)]
        i1_v[pl.ds(g * L, L)] = a + b

    sems = (sem_a, sem_b)
    lanes = lax.iota(jnp.int32, L)

    def fire(step, slot):
        s = sems[slot]
        for h in range(GC // L):
            iv0 = i0_v[pl.ds(step * GC + h * L, L)]
            iv1 = i1_v[pl.ds(step * GC + h * L, L)]
            for k2 in range(L):
                k = h * L + k2
                pltpu.async_copy(
                    tab_hbm.at[pl.ds(iv0[k2], 1), :], rows0_v.at[slot, k], s)
                pltpu.async_copy(
                    tab_hbm.at[pl.ds(iv1[k2], 1), :], rows1_v.at[slot, k], s)

    def drain(slot):
        pltpu.make_async_copy(
            tab_hbm.at[pl.ds(0, 2 * GC), :], drain_v, sems[slot]).wait()

    def compute(step, slot):
        for h in range(GC // L):
            acc = jnp.zeros((L,), jnp.float32)
            for k2 in range(L):
                k = h * L + k2
                a0 = rows0_v[slot, k, 0, pl.ds(0, L)]
                a1 = rows0_v[slot, k, 0, pl.ds(L, L)]
                b0 = rows1_v[slot, k, 0, pl.ds(0, L)]
                b1 = rows1_v[slot, k, 0, pl.ds(L, L)]
                s = jnp.sum(a0 * b0 + a1 * b1)
                acc = jnp.where(lanes == k2, s, acc)
            out_v[pl.ds(step * GC + h * L, L)] = acc

    fire(0, 0)
    fire(1, 1)

    def pair_body(j, _):
        step0 = 2 * j
        drain(0)
        compute(step0, 0)

        @pl.when(j < NSTEP // 2 - 1)
        def _():
            fire(step0 + 2, 0)

        drain(1)
        compute(step0 + 1, 1)

        @pl.when(j < NSTEP // 2 - 1)
        def _():
            fire(step0 + 3, 1)

        return 0
    lax.fori_loop(0, NSTEP // 2, pair_body, 0)

    pltpu.sync_copy(out_v, out_hbm.at[pl.ds(base, BPW)])


def kernel(x, table):
    return _sc_embed_dot(x.T, table.reshape(1, 1000000, D))


# speedup vs baseline: 1.0223x; 1.0223x over previous
import functools

import jax
import jax.numpy as jnp
from jax import lax
from jax.experimental import pallas as pl
from jax.experimental.pallas import tpu as pltpu
from jax.experimental.pallas import tpu_sc as plsc

NC = 2
NS = 16
L = 16
NW = NC * NS

B = 16384
D = 32
GC = 32
BPW = B // NW
NSTEP = BPW // GC

_mesh = plsc.VectorSubcoreMesh(core_axis_name="c", subcore_axis_name="s")


@functools.partial(
    pl.kernel,
    mesh=_mesh,
    compiler_params=pltpu.CompilerParams(
        needs_layout_passes=False, use_tc_tiling_on_sc=True),
    out_type=jax.ShapeDtypeStruct((B,), jnp.float32),
    scratch_types=[
        pltpu.VMEM((BPW,), jnp.int32),
        pltpu.VMEM((BPW,), jnp.int32),
        pltpu.VMEM((2, GC, 1, D), jnp.float32),
        pltpu.VMEM((2, GC, 1, D), jnp.float32),
        pltpu.VMEM((2 * GC, D), jnp.float32),
        pltpu.VMEM((BPW,), jnp.float32),
        pltpu.SemaphoreType.DMA,
        pltpu.SemaphoreType.DMA,
    ],
)
def _sc_embed_dot(xt_hbm, tab3_hbm, out_hbm,
                  i0_v, i1_v, rows0_v, rows1_v, drain_v, out_v, sem_a, sem_b):
    wid = lax.axis_index("s") * NC + lax.axis_index("c")
    base = pl.multiple_of(wid * BPW, 128)
    tab_hbm = tab3_hbm.at[0]

    pltpu.sync_copy(xt_hbm.at[0, pl.ds(base, BPW)], i0_v)
    pltpu.sync_copy(xt_hbm.at[1, pl.ds(base, BPW)], i1_v)

    for g in range(BPW // L):
        a = i0_v[pl.ds(g * L, L)]
        b = i1_v[pl.ds(g * L, L)]
        i1_v[pl.ds(g * L, L)] = a + b

    sems = (sem_a, sem_b)
    lanes = lax.iota(jnp.int32, L)

    def fire(step, slot):
        s = sems[slot]
        for h in range(GC // L):
            iv0 = i0_v[pl.ds(step * GC + h * L, L)]
            iv1 = i1_v[pl.ds(step * GC + h * L, L)]
            for k2 in range(L):
                k = h * L + k2
                pltpu.async_copy(
                    tab_hbm.at[pl.ds(iv0[k2], 1), :], rows0_v.at[slot, k], s)
                pltpu.async_copy(
                    tab_hbm.at[pl.ds(iv1[k2], 1), :], rows1_v.at[slot, k], s)

    def drain(slot):
        pltpu.make_async_copy(
            tab_hbm.at[pl.ds(0, 2 * GC), :], drain_v, sems[slot]).wait()

    def compute(step, slot):
        for h in range(GC // L):
            acc = jnp.zeros((L,), jnp.float32)
            for k2 in range(L):
                k = h * L + k2
                a0 = rows0_v[slot, k, 0, pl.ds(0, L)]
                a1 = rows0_v[slot, k, 0, pl.ds(L, L)]
                b0 = rows1_v[slot, k, 0, pl.ds(0, L)]
                b1 = rows1_v[slot, k, 0, pl.ds(L, L)]
                s = jnp.sum(a0 * b0 + a1 * b1)
                acc = jnp.where(lanes == k2, s, acc)
            out_v[pl.ds(step * GC + h * L, L)] = acc

    fire(0, 0)
    fire(1, 1)

    def pair_body(j, _):
        step0 = 2 * j
        drain(0)
        compute(step0, 0)

        @pl.when(j < NSTEP // 2 - 1)
        def _():
            fire(step0 + 2, 0)

        drain(1)
        compute(step0 + 1, 1)

        @pl.when(j < NSTEP // 2 - 1)
        def _():
            fire(step0 + 3, 1)

        return 0
    lax.fori_loop(0, NSTEP // 2, pair_body, 0)

    pltpu.sync_copy(out_v, out_hbm.at[pl.ds(base, BPW)])


def kernel(x, table):
    return _sc_embed_dot(x.T, table.reshape(1, 1000000, D))
